# baseline (speedup 1.0000x reference)
import functools

import jax
import jax.numpy as jnp
from jax import lax
from jax.experimental import pallas as pl
from jax.experimental.pallas import tpu as pltpu
from jax.experimental.pallas import tpu_sc as plsc

N = 10000
D = 128
E = 320000
NC = 2
NS = 16
NW = NC * NS
CHUNK = 64
NCHUNK = 160
HALF = NCHUNK // 2
E_W = NCHUNK * CHUNK
EPAD = NW * E_W
NP_ = 10240
ROWS_T = NP_ // NS
DEGW = 16

_mesh = plsc.VectorSubcoreMesh(core_axis_name="c", subcore_axis_name="s")


def _deg_count(dst3):
    @functools.partial(
        pl.kernel,
        out_type=jax.ShapeDtypeStruct((NC, NP_, DEGW), jnp.float32),
        mesh=_mesh,
        compiler_params=pltpu.CompilerParams(use_tc_tiling_on_sc=False),
        scratch_types=[
            pltpu.VMEM((NCHUNK, CHUNK), jnp.int32),
            pltpu.VMEM((CHUNK, DEGW), jnp.float32),
            pltpu.VMEM((CHUNK, DEGW), jnp.float32),
            pltpu.VMEM_SHARED((NP_, DEGW), jnp.float32),
        ],
    )
    def k(dst_hbm, deg_hbm, dst_v, zerod_v, ones_v, dega_sh):
        c = lax.axis_index("c")
        s = lax.axis_index("s")
        w = c * NS + s
        base = s * ROWS_T

        @pl.loop(0, CHUNK)
        def _(i):
            zerod_v[i, pl.ds(0, 16)] = jnp.zeros((16,), jnp.float32)
            ones_v[i, pl.ds(0, 16)] = jnp.full((16,), 1.0, jnp.float32)

        @pl.loop(0, ROWS_T // CHUNK)
        def _(k_):
            pltpu.sync_copy(zerod_v, dega_sh.at[pl.ds(base + k_ * CHUNK, CHUNK)])
        _rem = ROWS_T - (ROWS_T // CHUNK) * CHUNK
        if _rem:
            pltpu.sync_copy(zerod_v.at[pl.ds(0, _rem)],
                            dega_sh.at[pl.ds(base + ROWS_T - _rem, _rem)])

        pltpu.sync_copy(dst_hbm.at[w], dst_v)
        plsc.subcore_barrier()

        @pl.loop(0, NCHUNK)
        def _(j):
            pltpu.sync_copy(ones_v, dega_sh.at[dst_v.at[j]], add=True)

        plsc.subcore_barrier()
        pltpu.sync_copy(dega_sh.at[pl.ds(base, ROWS_T)],
                        deg_hbm.at[c, pl.ds(base, ROWS_T)])

    return k(dst3)


def _seg_sum(z, src3, dst3):
    @functools.partial(
        pl.kernel,
        out_type=jax.ShapeDtypeStruct((NC, NP_, D), jnp.float32),
        mesh=_mesh,
        compiler_params=pltpu.CompilerParams(use_tc_tiling_on_sc=False),
        scratch_types=[
            pltpu.VMEM((HALF, CHUNK), jnp.int32),
            pltpu.VMEM((HALF, CHUNK), jnp.int32),
            pltpu.VMEM((CHUNK, D), jnp.float32),
            pltpu.VMEM((CHUNK, D), jnp.float32),
            pltpu.VMEM((CHUNK, D), jnp.float32),
            pltpu.VMEM((CHUNK, D), jnp.float32),
            pltpu.VMEM_SHARED((NP_, D), jnp.float32),
        ] + [pltpu.SemaphoreType.DMA] * 8,
    )
    def k(z_hbm, src_hbm, dst_hbm, out_hbm,
          src_v, dst_v, r0, r1, r2, r3, acc_sh,
          g0, g1, g2, g3, s0, s1, s2, s3):
        c = lax.axis_index("c")
        s = lax.axis_index("s")
        w = c * NS + s
        base = s * ROWS_T
        bufs = [r0, r1, r2, r3]
        gsems = [g0, g1, g2, g3]
        ssems = [s0, s1, s2, s3]

        @pl.loop(0, CHUNK)
        def _(i):
            @pl.loop(0, D // 16)
            def _(j):
                r0[i, pl.ds(j * 16, 16)] = jnp.zeros((16,), jnp.float32)

        @pl.loop(0, ROWS_T // CHUNK)
        def _(k_):
            pltpu.sync_copy(r0, acc_sh.at[pl.ds(base + k_ * CHUNK, CHUNK)])

        def gather_start(j, bi):
            pltpu.async_copy(z_hbm.at[src_v.at[j]], bufs[bi], gsems[bi])

        def gather_wait(j, bi):
            pltpu.make_async_copy(z_hbm.at[src_v.at[j]], bufs[bi],
                                  gsems[bi]).wait()

        def scatter_start(j, bi):
            pltpu.async_copy(bufs[bi], acc_sh.at[dst_v.at[j]], ssems[bi],
                             add=True)

        def scatter_wait(j, bi):
            pltpu.make_async_copy(bufs[bi], acc_sh.at[dst_v.at[j]],
                                  ssems[bi]).wait()

        def stage_idx(h):
            pltpu.sync_copy(src_hbm.at[w, pl.ds(h * HALF, HALF)], src_v)
            pltpu.sync_copy(dst_hbm.at[w, pl.ds(h * HALF, HALF)], dst_v)

        def run_half():
            for m in range(4):
                gather_start(m, m)
            gather_wait(0, 0)
            scatter_start(0, 0)

            @pl.loop(0, (HALF - 4) // 4)
            def _(t):
                for r in range(4):
                    j = 4 * t + 1 + r
                    bi_a = r
                    bi_b = (1 + r) % 4
                    scatter_wait(j - 1, bi_a)
                    gather_start(j + 3, bi_a)
                    gather_wait(j, bi_b)
                    scatter_start(j, bi_b)

            for m in range(HALF - 3, HALF):
                gather_wait(m, m % 4)
                scatter_start(m, m % 4)
            for m in range(HALF - 4, HALF):
                scatter_wait(m, m % 4)

        stage_idx(0)
        plsc.subcore_barrier()
        run_half()
        stage_idx(1)
        run_half()

        plsc.subcore_barrier()
        pltpu.sync_copy(acc_sh.at[pl.ds(base, ROWS_T)],
                        out_hbm.at[c, pl.ds(base, ROWS_T)])

    return k(z, src3, dst3)


_RB = 1000


def _mm_body(x_ref, w_ref, o_ref):
    o_ref[...] = jnp.dot(x_ref[...], w_ref[...],
                         preferred_element_type=jnp.float32)


def _tc_matmul(x, w):
    return pl.pallas_call(
        _mm_body,
        grid=(N // _RB,),
        in_specs=[pl.BlockSpec((_RB, D), lambda i: (i, 0)),
                  pl.BlockSpec((D, D), lambda i: (0, 0))],
        out_specs=pl.BlockSpec((_RB, D), lambda i: (i, 0)),
        out_shape=jax.ShapeDtypeStruct((N, D), jnp.float32),
    )(x, w)


def _dinv_of(deg_ref):
    deg = deg_ref[0, :, :1] + deg_ref[1, :, :1]
    return 1.0 / jnp.maximum(deg, 1.0)


def _l1_body(agg_ref, deg_ref, b1_ref, w2_ref, wo1_ref, z2_ref, z3a_ref):
    agg = agg_ref[0] + agg_ref[1]
    h1 = jnp.maximum(agg * _dinv_of(deg_ref) + b1_ref[...], 0.0)
    z2_ref[...] = jnp.dot(h1, w2_ref[...], preferred_element_type=jnp.float32)
    z3a_ref[...] = jnp.dot(h1, wo1_ref[...], preferred_element_type=jnp.float32)


def _tc_layer1(agg1, deg, b1, W2, Wo1):
    return pl.pallas_call(
        _l1_body,
        grid=(N // _RB,),
        in_specs=[pl.BlockSpec((NC, _RB, D), lambda i: (0, i, 0)),
                  pl.BlockSpec((NC, _RB, DEGW), lambda i: (0, i, 0)),
                  pl.BlockSpec((1, D), lambda i: (0, 0)),
                  pl.BlockSpec((D, D), lambda i: (0, 0)),
                  pl.BlockSpec((D, D), lambda i: (0, 0))],
        out_specs=[pl.BlockSpec((_RB, D), lambda i: (i, 0)),
                   pl.BlockSpec((_RB, D), lambda i: (i, 0))],
        out_shape=[jax.ShapeDtypeStruct((N, D), jnp.float32),
                   jax.ShapeDtypeStruct((N, D), jnp.float32)],
    )(agg1, deg, b1, W2, Wo1)


def _l2_body(agg_ref, deg_ref, b2_ref, wo2_ref, z3a_ref, z3_ref):
    agg = agg_ref[0] + agg_ref[1]
    h2 = jnp.maximum(agg * _dinv_of(deg_ref) + b2_ref[...], 0.0)
    z3_ref[...] = z3a_ref[...] + jnp.dot(h2, wo2_ref[...],
                                         preferred_element_type=jnp.float32)


def _tc_layer2(agg2, deg, b2, Wo2, z3a):
    return pl.pallas_call(
        _l2_body,
        grid=(N // _RB,),
        in_specs=[pl.BlockSpec((NC, _RB, D), lambda i: (0, i, 0)),
                  pl.BlockSpec((NC, _RB, DEGW), lambda i: (0, i, 0)),
                  pl.BlockSpec((1, D), lambda i: (0, 0)),
                  pl.BlockSpec((D, D), lambda i: (0, 0)),
                  pl.BlockSpec((_RB, D), lambda i: (i, 0))],
        out_specs=pl.BlockSpec((_RB, D), lambda i: (i, 0)),
        out_shape=jax.ShapeDtypeStruct((N, D), jnp.float32),
    )(agg2, deg, b2, Wo2, z3a)


def _l3_body(agg_ref, deg_ref, bo_ref, o_ref):
    agg = agg_ref[0] + agg_ref[1]
    o_ref[...] = agg * _dinv_of(deg_ref) + bo_ref[...]


def _tc_layer3(agg3, deg, b_out):
    return pl.pallas_call(
        _l3_body,
        grid=(N // _RB,),
        in_specs=[pl.BlockSpec((NC, _RB, D), lambda i: (0, i, 0)),
                  pl.BlockSpec((NC, _RB, DEGW), lambda i: (0, i, 0)),
                  pl.BlockSpec((1, D), lambda i: (0, 0))],
        out_specs=pl.BlockSpec((_RB, D), lambda i: (i, 0)),
        out_shape=jax.ShapeDtypeStruct((N, D), jnp.float32),
    )(agg3, deg, b_out)


def kernel(feats, edge_index, W1, b1, W2, b2, W_out, b_out):
    ei = edge_index.astype(jnp.int32)
    pad = EPAD - E
    srcp = jnp.concatenate([ei[0], jnp.zeros((pad,), jnp.int32)])
    pad_dst = N + (jnp.arange(pad, dtype=jnp.int32) % (NP_ - N))
    dstp = jnp.concatenate([ei[1], pad_dst])
    src3 = srcp.reshape(NW, NCHUNK, CHUNK)
    dst3 = dstp.reshape(NW, NCHUNK, CHUNK)
    Wo1 = W_out[:D]
    Wo2 = W_out[D:]
    b1r = b1.reshape(1, D)
    b2r = b2.reshape(1, D)
    bor = b_out.reshape(1, D)

    deg = _deg_count(dst3)
    z1 = _tc_matmul(feats, W1)
    agg1 = _seg_sum(z1, src3, dst3)
    z2, z3a = _tc_layer1(agg1, deg, b1r, W2, Wo1)
    agg2 = _seg_sum(z2, src3, dst3)
    z3 = _tc_layer2(agg2, deg, b2r, Wo2, z3a)
    agg3 = _seg_sum(z3, src3, dst3)
    return _tc_layer3(agg3, deg, bor)


# speedup vs baseline: 3.3051x; 3.3051x over previous
import functools

import jax
import jax.numpy as jnp
from jax import lax
from jax.experimental import pallas as pl
from jax.experimental.pallas import tpu as pltpu
from jax.experimental.pallas import tpu_sc as plsc

N = 10000
D = 128
E = 320000
NC = 2
NS = 16
NW = NC * NS
CHUNK = 80
NCHUNK = 125
E_W = NCHUNK * CHUNK
EPAD = NW * E_W
NP_ = 10240
ROWS_T = NP_ // NS
DEGW = 16

_mesh = plsc.VectorSubcoreMesh(core_axis_name="c", subcore_axis_name="s")


def _deg_count(dst3):
    @functools.partial(
        pl.kernel,
        out_type=jax.ShapeDtypeStruct((NC, NP_, DEGW), jnp.float32),
        mesh=_mesh,
        compiler_params=pltpu.CompilerParams(use_tc_tiling_on_sc=False),
        scratch_types=[
            pltpu.VMEM((NCHUNK, CHUNK), jnp.int32),
            pltpu.VMEM((CHUNK, DEGW), jnp.float32),
            pltpu.VMEM((CHUNK, DEGW), jnp.float32),
            pltpu.VMEM_SHARED((NP_, DEGW), jnp.float32),
        ],
    )
    def k(dst_hbm, deg_hbm, dst_v, zerod_v, ones_v, dega_sh):
        c = lax.axis_index("c")
        s = lax.axis_index("s")
        w = c * NS + s
        base = s * ROWS_T

        @pl.loop(0, CHUNK)
        def _(i):
            zerod_v[i, pl.ds(0, 16)] = jnp.zeros((16,), jnp.float32)
            ones_v[i, pl.ds(0, 16)] = jnp.full((16,), 1.0, jnp.float32)

        @pl.loop(0, ROWS_T // CHUNK)
        def _(k_):
            pltpu.sync_copy(zerod_v, dega_sh.at[pl.ds(base + k_ * CHUNK, CHUNK)])
        _rem = ROWS_T - (ROWS_T // CHUNK) * CHUNK
        if _rem:
            pltpu.sync_copy(zerod_v.at[pl.ds(0, _rem)],
                            dega_sh.at[pl.ds(base + ROWS_T - _rem, _rem)])

        pltpu.sync_copy(dst_hbm.at[w], dst_v)
        plsc.subcore_barrier()

        @pl.loop(0, NCHUNK)
        def _(j):
            pltpu.sync_copy(ones_v, dega_sh.at[dst_v.at[j]], add=True)

        plsc.subcore_barrier()
        pltpu.sync_copy(dega_sh.at[pl.ds(base, ROWS_T)],
                        deg_hbm.at[c, pl.ds(base, ROWS_T)])

    return k(dst3)


def _seg_sum(z, src3, dst3):
    @functools.partial(
        pl.kernel,
        out_type=jax.ShapeDtypeStruct((NC, NP_, D), jnp.float32),
        mesh=_mesh,
        compiler_params=pltpu.CompilerParams(use_tc_tiling_on_sc=False),
        scratch_types=[
            pltpu.VMEM((NCHUNK, CHUNK), jnp.int32),
            pltpu.VMEM((NCHUNK, CHUNK), jnp.int32),
            pltpu.VMEM((CHUNK, D), jnp.float32),
            pltpu.VMEM((CHUNK, D), jnp.float32),
            pltpu.VMEM_SHARED((NP_, D), jnp.float32),
            pltpu.SemaphoreType.DMA,
            pltpu.SemaphoreType.DMA,
            pltpu.SemaphoreType.DMA,
            pltpu.SemaphoreType.DMA,
        ],
    )
    def k(z_hbm, src_hbm, dst_hbm, out_hbm,
          src_v, dst_v, r0, r1, acc_sh, sem0, sem1, sem2, sem3):
        c = lax.axis_index("c")
        s = lax.axis_index("s")
        w = c * NS + s
        base = s * ROWS_T

        @pl.loop(0, CHUNK)
        def _(i):
            @pl.loop(0, D // 16)
            def _(j):
                r0[i, pl.ds(j * 16, 16)] = jnp.zeros((16,), jnp.float32)

        @pl.loop(0, ROWS_T // CHUNK)
        def _(k_):
            pltpu.sync_copy(r0, acc_sh.at[pl.ds(base + k_ * CHUNK, CHUNK)])

        pltpu.sync_copy(src_hbm.at[w], src_v)
        pltpu.sync_copy(dst_hbm.at[w], dst_v)
        plsc.subcore_barrier()

        def gather_start(j, buf, sem):
            pltpu.async_copy(z_hbm.at[src_v.at[j]], buf, sem)

        def gather_wait(j, buf, sem):
            pltpu.make_async_copy(z_hbm.at[src_v.at[j]], buf, sem).wait()

        def scatter_start(j, buf, sem):
            pltpu.async_copy(buf, acc_sh.at[dst_v.at[j]], sem, add=True)

        def scatter_wait(j, buf, sem):
            pltpu.make_async_copy(buf, acc_sh.at[dst_v.at[j]], sem).wait()

        gather_start(0, r0, sem0)
        gather_start(1, r1, sem1)
        gather_wait(0, r0, sem0)
        scatter_start(0, r0, sem2)

        @pl.loop(1, NCHUNK - 2, step=2)
        def _(j):
            scatter_wait(j - 1, r0, sem2)
            gather_start(j + 1, r0, sem0)
            gather_wait(j, r1, sem1)
            scatter_start(j, r1, sem3)
            scatter_wait(j, r1, sem3)
            gather_start(j + 2, r1, sem1)
            gather_wait(j + 1, r0, sem0)
            scatter_start(j + 1, r0, sem2)

        gather_wait(NCHUNK - 2, r1, sem1)
        scatter_start(NCHUNK - 2, r1, sem3)
        scatter_wait(NCHUNK - 3, r0, sem2)
        gather_start(NCHUNK - 1, r0, sem0)
        gather_wait(NCHUNK - 1, r0, sem0)
        scatter_start(NCHUNK - 1, r0, sem2)
        scatter_wait(NCHUNK - 2, r1, sem3)
        scatter_wait(NCHUNK - 1, r0, sem2)

        plsc.subcore_barrier()
        pltpu.sync_copy(acc_sh.at[pl.ds(base, ROWS_T)],
                        out_hbm.at[c, pl.ds(base, ROWS_T)])

    return k(z, src3, dst3)


_RB = 1000


def _mm_body(x_ref, w_ref, o_ref):
    o_ref[...] = jnp.dot(x_ref[...], w_ref[...],
                         preferred_element_type=jnp.float32)


def _tc_matmul(x, w):
    return pl.pallas_call(
        _mm_body,
        grid=(N // _RB,),
        in_specs=[pl.BlockSpec((_RB, D), lambda i: (i, 0)),
                  pl.BlockSpec((D, D), lambda i: (0, 0))],
        out_specs=pl.BlockSpec((_RB, D), lambda i: (i, 0)),
        out_shape=jax.ShapeDtypeStruct((N, D), jnp.float32),
    )(x, w)


def _dinv_of(deg_ref):
    deg = deg_ref[0, :, :1] + deg_ref[1, :, :1]
    return 1.0 / jnp.maximum(deg, 1.0)


def _l1_body(agg_ref, deg_ref, b1_ref, w2_ref, wo1_ref, z2_ref, z3a_ref):
    agg = agg_ref[0] + agg_ref[1]
    h1 = jnp.maximum(agg * _dinv_of(deg_ref) + b1_ref[...], 0.0)
    z2_ref[...] = jnp.dot(h1, w2_ref[...], preferred_element_type=jnp.float32)
    z3a_ref[...] = jnp.dot(h1, wo1_ref[...], preferred_element_type=jnp.float32)


def _tc_layer1(agg1, deg, b1, W2, Wo1):
    return pl.pallas_call(
        _l1_body,
        grid=(N // _RB,),
        in_specs=[pl.BlockSpec((NC, _RB, D), lambda i: (0, i, 0)),
                  pl.BlockSpec((NC, _RB, DEGW), lambda i: (0, i, 0)),
                  pl.BlockSpec((1, D), lambda i: (0, 0)),
                  pl.BlockSpec((D, D), lambda i: (0, 0)),
                  pl.BlockSpec((D, D), lambda i: (0, 0))],
        out_specs=[pl.BlockSpec((_RB, D), lambda i: (i, 0)),
                   pl.BlockSpec((_RB, D), lambda i: (i, 0))],
        out_shape=[jax.ShapeDtypeStruct((N, D), jnp.float32),
                   jax.ShapeDtypeStruct((N, D), jnp.float32)],
    )(agg1, deg, b1, W2, Wo1)


def _l2_body(agg_ref, deg_ref, b2_ref, wo2_ref, z3a_ref, z3_ref):
    agg = agg_ref[0] + agg_ref[1]
    h2 = jnp.maximum(agg * _dinv_of(deg_ref) + b2_ref[...], 0.0)
    z3_ref[...] = z3a_ref[...] + jnp.dot(h2, wo2_ref[...],
                                         preferred_element_type=jnp.float32)


def _tc_layer2(agg2, deg, b2, Wo2, z3a):
    return pl.pallas_call(
        _l2_body,
        grid=(N // _RB,),
        in_specs=[pl.BlockSpec((NC, _RB, D), lambda i: (0, i, 0)),
                  pl.BlockSpec((NC, _RB, DEGW), lambda i: (0, i, 0)),
                  pl.BlockSpec((1, D), lambda i: (0, 0)),
                  pl.BlockSpec((D, D), lambda i: (0, 0)),
                  pl.BlockSpec((_RB, D), lambda i: (i, 0))],
        out_specs=pl.BlockSpec((_RB, D), lambda i: (i, 0)),
        out_shape=jax.ShapeDtypeStruct((N, D), jnp.float32),
    )(agg2, deg, b2, Wo2, z3a)


def _l3_body(agg_ref, deg_ref, bo_ref, o_ref):
    agg = agg_ref[0] + agg_ref[1]
    o_ref[...] = agg * _dinv_of(deg_ref) + bo_ref[...]


def _tc_layer3(agg3, deg, b_out):
    return pl.pallas_call(
        _l3_body,
        grid=(N // _RB,),
        in_specs=[pl.BlockSpec((NC, _RB, D), lambda i: (0, i, 0)),
                  pl.BlockSpec((NC, _RB, DEGW), lambda i: (0, i, 0)),
                  pl.BlockSpec((1, D), lambda i: (0, 0))],
        out_specs=pl.BlockSpec((_RB, D), lambda i: (i, 0)),
        out_shape=jax.ShapeDtypeStruct((N, D), jnp.float32),
    )(agg3, deg, b_out)


def kernel(feats, edge_index, W1, b1, W2, b2, W_out, b_out):
    ei = edge_index.astype(jnp.int32)
    pad = EPAD - E
    srcp = jnp.concatenate([ei[0], jnp.zeros((pad,), jnp.int32)])
    pad_dst = N + (jnp.arange(pad, dtype=jnp.int32) % (NP_ - N))
    dstp = jnp.concatenate([ei[1], pad_dst])
    src3 = srcp.reshape(NW, NCHUNK, CHUNK)
    dst3 = dstp.reshape(NW, NCHUNK, CHUNK)
    Wo1 = W_out[:D]
    Wo2 = W_out[D:]
    b1r = b1.reshape(1, D)
    b2r = b2.reshape(1, D)
    bor = b_out.reshape(1, D)

    deg = _deg_count(dst3)
    z1 = _tc_matmul(feats, W1)
    agg1 = _seg_sum(z1, src3, dst3)
    z2, z3a = _tc_layer1(agg1, deg, b1r, W2, Wo1)
    agg2 = _seg_sum(z2, src3, dst3)
    z3 = _tc_layer2(agg2, deg, b2r, Wo2, z3a)
    agg3 = _seg_sum(z3, src3, dst3)
    return _tc_layer3(agg3, deg, bor)
